# Initial kernel scaffold; baseline (speedup 1.0000x reference)
import jax
import jax.numpy as jnp
from jax.experimental import pallas as pl


def kernel(hidden_states, Wqkv, bqkv, Wr1, br1, Wr2, br2):
    raise NotImplementedError("write your pallas kernel here")


# speedup vs baseline: 2.5983x; 2.5983x over previous
import jax
import jax.numpy as jnp
from jax import lax
from jax.experimental import pallas as pl

B, T, E, H = 16, 576, 1024, 16
HD = E // H
K_SEL = 230
NEG_INF = float("-inf")


def _proj_kernel(h_ref, wqkv_ref, bqkv_ref, wr1_ref, br1_ref, wr2_ref, br2_ref,
                 qkv_ref, s_ref):
    h = h_ref[0]
    acc = jnp.dot(h, wqkv_ref[...], preferred_element_type=jnp.float32)
    qkv_ref[0] = acc + bqkv_ref[...]
    r1 = jnp.dot(h, wr1_ref[...], preferred_element_type=jnp.float32) + br1_ref[...]
    r1 = jnp.maximum(r1, 0.0)
    z = jnp.dot(r1, wr2_ref[...], preferred_element_type=jnp.float32) + br2_ref[...]
    s_ref[0] = jax.nn.sigmoid(jax.nn.sigmoid(z))


def _attn_kernel(q_ref, k_ref, v_ref, srow_ref, scol_ref, o_ref):
    q2 = q_ref[0]
    k2 = k_ref[0]
    v2 = v_ref[0]
    srow2 = srow_ref[0, 0]
    scol2 = scol_ref[0, 0]
    row_lt_col = lax.broadcasted_iota(jnp.int32, (T, T), 0) < lax.broadcasted_iota(
        jnp.int32, (T, T), 1)
    for j in range(2):
        srow = srow2[j:j + 1, :]
        scol = scol2[:, j:j + 1]
        sj = jnp.broadcast_to(scol, (T, T))
        si = jnp.broadcast_to(srow, (T, T))
        contrib = jnp.where(sj > si, 1.0, 0.0) + jnp.where(
            (sj == si) & row_lt_col, 1.0, 0.0)
        rank = jnp.sum(contrib, axis=0, keepdims=True)
        maskadd = jnp.where(rank < float(K_SEL), 0.0, NEG_INF)
        q = q2[:, j * HD:(j + 1) * HD]
        k = k2[:, j * HD:(j + 1) * HD]
        v = v2[:, j * HD:(j + 1) * HD]
        logits = lax.dot_general(q, k, (((1,), (1,)), ((), ())),
                                 preferred_element_type=jnp.float32)
        logits = logits * (1.0 / 8.0) + maskadd
        m = jnp.max(logits, axis=1, keepdims=True)
        p = jnp.exp(logits - m)
        w = p / jnp.sum(p, axis=1, keepdims=True)
        o_ref[0, :, j * HD:(j + 1) * HD] = jnp.dot(
            w, v, preferred_element_type=jnp.float32)


@jax.jit
def kernel(hidden_states, Wqkv, bqkv, Wr1, br1, Wr2, br2):
    f32 = jnp.float32
    qkv, scores = pl.pallas_call(
        _proj_kernel,
        grid=(B,),
        in_specs=[
            pl.BlockSpec((1, T, E), lambda i: (i, 0, 0)),
            pl.BlockSpec((E, 3 * E), lambda i: (0, 0)),
            pl.BlockSpec((1, 3 * E), lambda i: (0, 0)),
            pl.BlockSpec((E, E // 4), lambda i: (0, 0)),
            pl.BlockSpec((1, E // 4), lambda i: (0, 0)),
            pl.BlockSpec((E // 4, H), lambda i: (0, 0)),
            pl.BlockSpec((1, H), lambda i: (0, 0)),
        ],
        out_specs=[
            pl.BlockSpec((1, T, 3 * E), lambda i: (i, 0, 0)),
            pl.BlockSpec((1, T, H), lambda i: (i, 0, 0)),
        ],
        out_shape=[
            jax.ShapeDtypeStruct((B, T, 3 * E), f32),
            jax.ShapeDtypeStruct((B, T, H), f32),
        ],
    )(hidden_states, Wqkv, bqkv.reshape(1, 3 * E), Wr1, br1.reshape(1, E // 4),
      Wr2, br2.reshape(1, H))

    G = H // 2
    s_t = jnp.transpose(scores, (0, 2, 1))
    srow4 = s_t.reshape(B, G, 2, T)
    scol4 = jnp.transpose(srow4, (0, 1, 3, 2))

    out = pl.pallas_call(
        _attn_kernel,
        grid=(B, G),
        in_specs=[
            pl.BlockSpec((1, T, 2 * HD), lambda b, g: (b, 0, g)),
            pl.BlockSpec((1, T, 2 * HD), lambda b, g: (b, 0, G + g)),
            pl.BlockSpec((1, T, 2 * HD), lambda b, g: (b, 0, 2 * G + g)),
            pl.BlockSpec((1, 1, 2, T), lambda b, g: (b, g, 0, 0)),
            pl.BlockSpec((1, 1, T, 2), lambda b, g: (b, g, 0, 0)),
        ],
        out_specs=pl.BlockSpec((1, T, 2 * HD), lambda b, g: (b, 0, g)),
        out_shape=jax.ShapeDtypeStruct((B, T, E), f32),
    )(qkv, qkv, qkv, srow4, scol4)

    return out
